# baseline (speedup 1.0000x reference)
import functools

import jax
import jax.numpy as jnp
from jax import lax
from jax.experimental import pallas as pl
from jax.experimental.pallas import tpu as pltpu
from jax.experimental.pallas import tpu_sc as plsc

NC = 2
NS = 16
NW = NC * NS
L = 16


@functools.lru_cache(maxsize=None)
def _build(b, v1, d, total):
    vis = v1 - 1
    nb = b // NW
    t1 = total + 1
    n_chunk = total // L

    mesh = plsc.VectorSubcoreMesh(core_axis_name="c", subcore_axis_name="s")

    @functools.partial(
        pl.kernel,
        mesh=mesh,
        out_type=jax.ShapeDtypeStruct((b, t1 * d), jnp.float32),
        scratch_types=[
            pltpu.VMEM((nb, total), jnp.int32),
            pltpu.VMEM(((v1 + 1) * d,), jnp.float32),
            pltpu.VMEM(((v1 + 1) * d,), jnp.float32),
            pltpu.VMEM((t1 * d,), jnp.float32),
            pltpu.VMEM((t1 * d,), jnp.float32),
            pltpu.SemaphoreType.DMA,
            pltpu.SemaphoreType.DMA,
            pltpu.SemaphoreType.DMA,
            pltpu.SemaphoreType.DMA,
        ],
        compiler_params=pltpu.CompilerParams(use_tc_tiling_on_sc=False,
                                             needs_layout_passes=False),
    )
    def k(img_hbm, mask_hbm, idx_hbm, out_hbm, idx_all, img_a, img_b,
          out_a, out_b, isem_a, isem_b, osem_a, osem_b):
        wid = lax.axis_index("s") * NC + lax.axis_index("c")
        b0 = wid * nb
        cpi = pltpu.async_copy(idx_hbm.at[pl.ds(b0, nb)], idx_all, osem_a)
        cpm_a = pltpu.async_copy(mask_hbm, img_a.at[pl.ds(v1 * d, d)],
                                 osem_a)
        cpm_b = pltpu.async_copy(mask_hbm, img_b.at[pl.ds(v1 * d, d)],
                                 osem_a)
        cpi.wait()
        cpm_a.wait()
        cpm_b.wait()
        cols = [lax.broadcasted_iota(jnp.int32, (L,), 0) + cc * L
                for cc in range(d // L)]
        lanes = [jnp.full((L,), j, jnp.int32) for j in range(L)]

        def start_img(bb, buf, sem):
            pltpu.async_copy(img_hbm.at[bb], buf.at[pl.ds(0, v1 * d)], sem)

        def wait_img(buf, sem):
            pltpu.make_async_copy(img_hbm.at[0], buf.at[pl.ds(0, v1 * d)],
                                  sem).wait()

        def wait_out(buf, sem):
            pltpu.make_async_copy(buf, out_hbm.at[0], sem).wait()

        def assemble(i, img_buf, out_buf, lo, hi):
            if lo == 0:
                for cc in range(d // L):
                    out_buf[pl.ds(cc * L, L)] = img_buf[pl.ds(cc * L, L)]

            @plsc.parallel_loop(lo, hi)
            def cbody(c):
                v = idx_all[i, pl.ds(c * L, L)]
                lr96 = jnp.where(v < vis, v + 1, v1) * d
                base = (c * L + 1) * d
                for j in range(L):
                    rbj = lax.gather(
                        lr96, lanes[j][:, None],
                        lax.GatherDimensionNumbers(
                            offset_dims=(), collapsed_slice_dims=(0,),
                            start_index_map=(0,)),
                        slice_sizes=(1,),
                        mode=lax.GatherScatterMode.PROMISE_IN_BOUNDS)
                    for cc in range(d // L):
                        val = plsc.load_gather(img_buf, [rbj + cols[cc]])
                        out_buf[pl.ds(base + (j * d + cc * L), L)] = val

        start_img(b0, img_a, isem_a)

        def body(g, carry):
            for sl in range(2):
                i = 2 * g + sl
                bb = b0 + i
                img_buf = img_a if sl == 0 else img_b
                out_buf = out_a if sl == 0 else out_b
                isem = isem_a if sl == 0 else isem_b
                osem = osem_a if sl == 0 else osem_b
                nxt_buf = img_b if sl == 0 else img_a
                nxt_sem = isem_b if sl == 0 else isem_a

                wait_img(img_buf, isem)

                @pl.when(i + 1 < nb)
                def _():
                    start_img(bb + 1, nxt_buf, nxt_sem)

                @pl.when(i >= 2)
                def _():
                    wait_out(out_buf, osem)

                h0 = (n_chunk // 2 * L + 1) * d
                assemble(i, img_buf, out_buf, 0, n_chunk // 2)
                pltpu.async_copy(out_buf.at[pl.ds(0, h0)],
                                 out_hbm.at[bb, pl.ds(0, h0)], osem)
                assemble(i, img_buf, out_buf, n_chunk // 2, n_chunk)
                pltpu.async_copy(out_buf.at[pl.ds(h0, t1 * d - h0)],
                                 out_hbm.at[bb, pl.ds(h0, t1 * d - h0)],
                                 osem)
            return carry

        lax.fori_loop(0, nb // 2, body, 0)
        wait_out(out_a, osem_a)
        wait_out(out_b, osem_b)

    return k


def kernel(img, img_revert_idx, mask_token):
    b, v1, d = img.shape
    total = img_revert_idx.shape[1]
    out = _build(b, v1, d, total)(
        img.reshape(b, v1 * d), mask_token.reshape(d), img_revert_idx)
    return out.reshape(b, total + 1, d)


# speedup vs baseline: 1.1706x; 1.1706x over previous
import functools

import jax
import jax.numpy as jnp
from jax import lax
from jax.experimental import pallas as pl
from jax.experimental.pallas import tpu as pltpu
from jax.experimental.pallas import tpu_sc as plsc

NC = 2
NS = 16
NW = NC * NS
L = 16


@functools.lru_cache(maxsize=None)
def _build(b, v1, d, total):
    vis = v1 - 1
    nb = b // NW
    t1 = total + 1
    n_chunk = total // L

    mesh = plsc.VectorSubcoreMesh(core_axis_name="c", subcore_axis_name="s")

    @functools.partial(
        pl.kernel,
        mesh=mesh,
        out_type=jax.ShapeDtypeStruct((b, t1 * d), jnp.float32),
        scratch_types=[
            pltpu.VMEM((nb, total), jnp.int32),
            pltpu.VMEM(((v1 + 1) * d,), jnp.float32),
            pltpu.VMEM(((v1 + 1) * d,), jnp.float32),
            pltpu.VMEM((t1 * d,), jnp.float32),
            pltpu.VMEM((t1 * d,), jnp.float32),
            pltpu.SemaphoreType.DMA,
            pltpu.SemaphoreType.DMA,
            pltpu.SemaphoreType.DMA,
            pltpu.SemaphoreType.DMA,
        ],
        compiler_params=pltpu.CompilerParams(use_tc_tiling_on_sc=False,
                                             needs_layout_passes=False),
    )
    def k(img_hbm, mask_hbm, idx_hbm, out_hbm, idx_all, img_a, img_b,
          out_a, out_b, isem_a, isem_b, osem_a, osem_b):
        wid = lax.axis_index("s") * NC + lax.axis_index("c")
        b0 = wid * nb
        cpi = pltpu.async_copy(idx_hbm.at[pl.ds(b0, nb)], idx_all, osem_a)
        cpm_a = pltpu.async_copy(mask_hbm, img_a.at[pl.ds(v1 * d, d)],
                                 osem_a)
        cpm_b = pltpu.async_copy(mask_hbm, img_b.at[pl.ds(v1 * d, d)],
                                 osem_a)
        cpi.wait()
        cpm_a.wait()
        cpm_b.wait()
        cols = [lax.broadcasted_iota(jnp.int32, (L,), 0) + cc * L
                for cc in range(d // L)]
        lanes = [jnp.full((L,), j, jnp.int32) for j in range(L)]

        def start_img(bb, buf, sem):
            pltpu.async_copy(img_hbm.at[bb], buf.at[pl.ds(0, v1 * d)], sem)

        def wait_img(buf, sem):
            pltpu.make_async_copy(img_hbm.at[0], buf.at[pl.ds(0, v1 * d)],
                                  sem).wait()

        def wait_out(buf, sem):
            pltpu.make_async_copy(buf, out_hbm.at[0], sem).wait()

        def assemble(i, img_buf, out_buf):
            for cc in range(d // L):
                out_buf[pl.ds(cc * L, L)] = img_buf[pl.ds(cc * L, L)]

            @plsc.parallel_loop(0, n_chunk)
            def cbody(c):
                v = idx_all[i, pl.ds(c * L, L)]
                lr96 = jnp.where(v < vis, v + 1, v1) * d
                base = (c * L + 1) * d
                for j in range(L):
                    rbj = lax.gather(
                        lr96, lanes[j][:, None],
                        lax.GatherDimensionNumbers(
                            offset_dims=(), collapsed_slice_dims=(0,),
                            start_index_map=(0,)),
                        slice_sizes=(1,),
                        mode=lax.GatherScatterMode.PROMISE_IN_BOUNDS)
                    for cc in range(d // L):
                        val = plsc.load_gather(img_buf, [rbj + cols[cc]])
                        out_buf[pl.ds(base + (j * d + cc * L), L)] = val

        start_img(b0, img_a, isem_a)

        def body(g, carry):
            for sl in range(2):
                i = 2 * g + sl
                bb = b0 + i
                img_buf = img_a if sl == 0 else img_b
                out_buf = out_a if sl == 0 else out_b
                isem = isem_a if sl == 0 else isem_b
                osem = osem_a if sl == 0 else osem_b
                nxt_buf = img_b if sl == 0 else img_a
                nxt_sem = isem_b if sl == 0 else isem_a

                wait_img(img_buf, isem)

                @pl.when(i + 1 < nb)
                def _():
                    start_img(bb + 1, nxt_buf, nxt_sem)

                @pl.when(i >= 2)
                def _():
                    wait_out(out_buf, osem)

                assemble(i, img_buf, out_buf)
                pltpu.async_copy(out_buf, out_hbm.at[bb], osem)
            return carry

        lax.fori_loop(0, nb // 2, body, 0)
        wait_out(out_a, osem_a)
        wait_out(out_b, osem_b)

    return k


def kernel(img, img_revert_idx, mask_token):
    b, v1, d = img.shape
    total = img_revert_idx.shape[1]
    out = _build(b, v1, d, total)(
        img.reshape(b, v1 * d), mask_token.reshape(d), img_revert_idx)
    return out.reshape(b, total + 1, d)
